# baseline (speedup 1.0000x reference)
import functools

import jax
import jax.numpy as jnp
from jax import lax
from jax.experimental import pallas as pl
from jax.experimental.pallas import tpu as pltpu
from jax.experimental.pallas import tpu_sc as plsc

N = 10000
E = 320000
D_IN = 128
D_EDGE = 16
LATENT = 64

NC = 2
NS = 16
NW = NC * NS
CB = 96
G0 = 148
G1 = 64
E_PAD = NS * (G0 + G1) * CB
N_PAD = 10112
ROWS_PER_TILE = N_PAD // NS
WB_CHUNKS = [(j * CB, CB) for j in range(ROWS_PER_TILE // CB)]
if ROWS_PER_TILE % CB:
    WB_CHUNKS.append((ROWS_PER_TILE - ROWS_PER_TILE % CB, ROWS_PER_TILE % CB))


def _edge_embed_body(ea_ref, w_ref, b_ref, e_ref):
    e_ref[...] = (jnp.dot(ea_ref[...], w_ref[...], preferred_element_type=jnp.float32)
                  + b_ref[...])


def _edge_embed(ea_pad, wt, b):
    BE = 2048
    grid = (E_PAD // BE,)
    full = lambda shape: pl.BlockSpec(shape, lambda i: (0, 0))
    return pl.pallas_call(
        _edge_embed_body,
        grid=grid,
        in_specs=[
            pl.BlockSpec((BE, D_EDGE), lambda i: (i, 0)),
            full((D_EDGE, D_IN)), full((1, D_IN)),
        ],
        out_specs=pl.BlockSpec((BE, D_IN), lambda i: (i, 0)),
        out_shape=jax.ShapeDtypeStruct((E_PAD, D_IN), jnp.float32),
    )(ea_pad, wt, b)


def _zero_buf(buf, rows, d):
    z = jnp.zeros((16,), jnp.float32)

    def row(r, _):
        for c in range(d // 16):
            buf[r, pl.ds(c * 16, 16)] = z
        return 0

    lax.fori_loop(0, rows, row, 0, unroll=False)


def _relu_add(dst, a, b, rows, d):
    def row(r, _):
        for c in range(d // 16):
            sl = pl.ds(c * 16, 16)
            dst[r, sl] = jnp.maximum(a[r, sl] + b[r, sl], 0.0)
        return 0

    lax.fori_loop(0, rows, row, 0, unroll=False)


def _sc_aggregate(table, src, dst, e):
    mesh = plsc.VectorSubcoreMesh(core_axis_name="c", subcore_axis_name="s",
                                  num_cores=NC, num_subcores=NS)

    @functools.partial(
        pl.kernel,
        out_type=jax.ShapeDtypeStruct((NC, N_PAD, D_IN), jnp.float32),
        mesh=mesh,
        scratch_types=[
            pltpu.VMEM((CB,), jnp.int32), pltpu.VMEM((CB,), jnp.int32),
            pltpu.VMEM((CB,), jnp.int32), pltpu.VMEM((CB,), jnp.int32),
            pltpu.VMEM((CB, D_IN), jnp.float32), pltpu.VMEM((CB, D_IN), jnp.float32),
            pltpu.VMEM((CB, D_IN), jnp.float32), pltpu.VMEM((CB, D_IN), jnp.float32),
            pltpu.VMEM_SHARED((N_PAD, D_IN), jnp.float32),
            pltpu.SemaphoreType.DMA, pltpu.SemaphoreType.DMA,
            pltpu.SemaphoreType.DMA, pltpu.SemaphoreType.DMA,
            pltpu.SemaphoreType.DMA, pltpu.SemaphoreType.DMA,
        ],
    )
    def k(tab_hbm, src_hbm, dst_hbm, e_hbm, out_hbm,
          sidx0, sidx1, didx0, didx1, rows0, rows1, e0, e1, acc_sh,
          sem_s0, sem_s1, sem_in0, sem_in1, sem_g0, sem_g1):
        cid = lax.axis_index("c")
        sid = lax.axis_index("s")
        gc = jnp.where(cid == 0, G0, G1)
        ebase = jnp.where(cid == 0, sid * (G0 * CB),
                          NS * (G0 * CB) + sid * (G1 * CB))

        sidx = (sidx0, sidx1)
        didx = (didx0, didx1)
        rows = (rows0, rows1)
        ebuf = (e0, e1)
        sem_s = (sem_s0, sem_s1)
        sem_in = (sem_in0, sem_in1)
        sem_g = (sem_g0, sem_g1)

        def issue_sidx(g, b):
            base = ebase + g * CB
            pltpu.async_copy(src_hbm.at[pl.ds(base, CB)], sidx[b], sem_s[b])

        def issue_de(g, b):
            base = ebase + g * CB
            pltpu.async_copy(dst_hbm.at[pl.ds(base, CB)], didx[b], sem_in[b])
            pltpu.async_copy(e_hbm.at[pl.ds(base, CB), :], ebuf[b], sem_in[b])

        def wait_sidx(b):
            pltpu.make_async_copy(src_hbm.at[pl.ds(0, CB)], sidx[b], sem_s[b]).wait()

        def issue_gather(b):
            pltpu.async_copy(tab_hbm.at[sidx[b]], rows[b], sem_g[b])

        def wait_gather(b):
            pltpu.make_async_copy(tab_hbm.at[sidx[b]], rows[b], sem_g[b]).wait()

        def wait_in(b):
            pltpu.make_async_copy(dst_hbm.at[pl.ds(0, CB)], didx[b], sem_in[b]).wait()
            pltpu.make_async_copy(e_hbm.at[pl.ds(0, CB), :], ebuf[b], sem_in[b]).wait()

        _zero_buf(rows0, CB, D_IN)
        row0 = sid * ROWS_PER_TILE
        for off, nr in WB_CHUNKS:
            pltpu.sync_copy(rows0.at[pl.ds(0, nr), :],
                            acc_sh.at[pl.ds(row0 + off, nr), :])
        plsc.subcore_barrier()

        issue_sidx(0, 0)
        issue_de(0, 0)
        issue_sidx(1, 1)
        issue_de(1, 1)
        wait_sidx(0)
        issue_gather(0)

        def step(t, _):
            for b in range(2):
                g = 2 * t + b
                nb = 1 - b

                @pl.when(g + 1 < gc)
                def _():
                    wait_sidx(nb)
                    issue_gather(nb)

                wait_gather(b)

                @pl.when(g + 2 < gc)
                def _():
                    issue_sidx(g + 2, b)

                wait_in(b)
                _relu_add(rows[b], rows[b], ebuf[b], CB, D_IN)
                pltpu.sync_copy(rows[b], acc_sh.at[didx[b]], add=True)

                @pl.when(g + 2 < gc)
                def _():
                    issue_de(g + 2, b)
            return 0

        lax.fori_loop(0, gc // 2, step, 0, unroll=False)
        plsc.subcore_barrier()

        for off, nr in WB_CHUNKS:
            sl = pl.ds(row0 + off, nr)
            st = pl.ds(0, nr)
            pltpu.sync_copy(acc_sh.at[sl, :], rows0.at[st, :])
            pltpu.sync_copy(rows0.at[st, :], out_hbm.at[cid, sl, :])

    return k(table, src, dst, e)


def _mlp1_body(x_ref, a0_ref, a1_ref, w1_ref, b1_ref, w2_ref, b2_ref, hh_ref):
    s = x_ref[...] + a0_ref[0] + a1_ref[0]
    t = jnp.maximum(jnp.dot(s, w1_ref[...], preferred_element_type=jnp.float32)
                    + b1_ref[...], 0.0)
    u = jnp.dot(t, w2_ref[...], preferred_element_type=jnp.float32) + b2_ref[...]
    h = jnp.maximum(u, 0.0)
    hh_ref[...] = jnp.concatenate([h, h], axis=1)


def _mlp1(x, acc, w1t, b1, w2t, b2):
    BN = 1000
    grid = (N // BN,)
    full = lambda shape: pl.BlockSpec(shape, lambda i: (0, 0))
    return pl.pallas_call(
        _mlp1_body,
        grid=grid,
        in_specs=[
            pl.BlockSpec((BN, D_IN), lambda i: (i, 0)),
            pl.BlockSpec((1, BN, D_IN), lambda i: (0, i, 0)),
            pl.BlockSpec((1, BN, D_IN), lambda i: (1, i, 0)),
            full((D_IN, LATENT)), full((1, LATENT)),
            full((LATENT, LATENT)), full((1, LATENT)),
        ],
        out_specs=pl.BlockSpec((BN, D_IN), lambda i: (i, 0)),
        out_shape=jax.ShapeDtypeStruct((N, D_IN), jnp.float32),
    )(x, acc, acc, w1t, b1, w2t, b2)


def _mlp23_body(hh_ref, a0_ref, a1_ref,
                mw1_ref, mb1_ref, mw2_ref, mb2_ref,
                lw1_ref, lb1_ref, lw2_ref, lb2_ref, mu_ref, ls_ref):
    h = hh_ref[:, :LATENT]
    a = a0_ref[0] + a1_ref[0]
    s2 = h + a[:, :LATENT]
    t2 = jnp.maximum(jnp.dot(s2, mw1_ref[...], preferred_element_type=jnp.float32)
                     + mb1_ref[...], 0.0)
    mu_ref[...] = jnp.dot(t2, mw2_ref[...], preferred_element_type=jnp.float32) + mb2_ref[...]
    s3 = h + a[:, LATENT:]
    t3 = jnp.maximum(jnp.dot(s3, lw1_ref[...], preferred_element_type=jnp.float32)
                     + lb1_ref[...], 0.0)
    u3 = jnp.dot(t3, lw2_ref[...], preferred_element_type=jnp.float32) + lb2_ref[...]
    ls_ref[...] = jnp.clip(u3, -10.0, 10.0)


def _mlp23(hh, acc, mw1t, mb1, mw2t, mb2, lw1t, lb1, lw2t, lb2):
    BN = 1000
    grid = (N // BN,)
    full = lambda shape: pl.BlockSpec(shape, lambda i: (0, 0))
    return pl.pallas_call(
        _mlp23_body,
        grid=grid,
        in_specs=[
            pl.BlockSpec((BN, D_IN), lambda i: (i, 0)),
            pl.BlockSpec((1, BN, D_IN), lambda i: (0, i, 0)),
            pl.BlockSpec((1, BN, D_IN), lambda i: (1, i, 0)),
            full((LATENT, LATENT)), full((1, LATENT)),
            full((LATENT, LATENT)), full((1, LATENT)),
            full((LATENT, LATENT)), full((1, LATENT)),
            full((LATENT, LATENT)), full((1, LATENT)),
        ],
        out_specs=[
            pl.BlockSpec((BN, LATENT), lambda i: (i, 0)),
            pl.BlockSpec((BN, LATENT), lambda i: (i, 0)),
        ],
        out_shape=[
            jax.ShapeDtypeStruct((N, LATENT), jnp.float32),
            jax.ShapeDtypeStruct((N, LATENT), jnp.float32),
        ],
    )(hh, acc, acc, mw1t, mb1, mw2t, mb2, lw1t, lb1, lw2t, lb2)


def kernel(x, edge_index, edge_attr,
           le1_W, le1_b, c1_W1, c1_b1, c1_W2, c1_b2,
           le2_W, le2_b, mu_W1, mu_b1, mu_W2, mu_b2,
           le3_W, le3_b, ls_W1, ls_b1, ls_W2, ls_b2):
    src = edge_index[0].astype(jnp.int32)
    dst = edge_index[1].astype(jnp.int32)
    pad = E_PAD - E
    src = jnp.pad(src, (0, pad))
    dst = jnp.pad(dst, (0, pad), constant_values=N)
    ea = jnp.pad(edge_attr.reshape(-1), (0, pad * D_EDGE)).reshape(E_PAD, D_EDGE)

    w23t = jnp.concatenate([le2_W.T, le3_W.T], axis=1)
    b23 = jnp.concatenate([le2_b, le3_b]).reshape(1, -1)
    e1 = _edge_embed(ea, le1_W.T, le1_b.reshape(1, -1))

    acc1 = _sc_aggregate(x, src, dst, e1)
    e23 = _edge_embed(ea, w23t, b23)
    hh = _mlp1(x, acc1, c1_W1.T, c1_b1.reshape(1, -1), c1_W2.T, c1_b2.reshape(1, -1))
    acc23 = _sc_aggregate(hh, src, dst, e23)
    mu, logstd = _mlp23(
        hh, acc23,
        mu_W1.T, mu_b1.reshape(1, -1), mu_W2.T, mu_b2.reshape(1, -1),
        ls_W1.T, ls_b1.reshape(1, -1), ls_W2.T, ls_b2.reshape(1, -1),
    )
    return (mu, logstd)


# speedup vs baseline: 1.0852x; 1.0852x over previous
import functools

import jax
import jax.numpy as jnp
from jax import lax
from jax.experimental import pallas as pl
from jax.experimental.pallas import tpu as pltpu
from jax.experimental.pallas import tpu_sc as plsc

N = 10000
E = 320000
D_IN = 128
D_EDGE = 16
LATENT = 64

NC = 2
NS = 16
NW = NC * NS
CB = 96
G0 = 148
G1 = 64
E_PAD = NS * (G0 + G1) * CB
N_PAD = 10112
ROWS_PER_TILE = N_PAD // NS
WB_CHUNKS = [(j * CB, CB) for j in range(ROWS_PER_TILE // CB)]
if ROWS_PER_TILE % CB:
    WB_CHUNKS.append((ROWS_PER_TILE - ROWS_PER_TILE % CB, ROWS_PER_TILE % CB))


def _edge_embed_body(ea_ref, w_ref, b_ref, e_ref):
    e_ref[...] = (jnp.dot(ea_ref[...], w_ref[...], preferred_element_type=jnp.float32)
                  + b_ref[...])


def _edge_embed(ea_pad, wt, b):
    BE = 2048
    grid = (E_PAD // BE,)
    full = lambda shape: pl.BlockSpec(shape, lambda i: (0, 0))
    return pl.pallas_call(
        _edge_embed_body,
        grid=grid,
        in_specs=[
            pl.BlockSpec((BE, D_EDGE), lambda i: (i, 0)),
            full((D_EDGE, D_IN)), full((1, D_IN)),
        ],
        out_specs=pl.BlockSpec((BE, D_IN), lambda i: (i, 0)),
        out_shape=jax.ShapeDtypeStruct((E_PAD, D_IN), jnp.float32),
    )(ea_pad, wt, b)


def _zero_buf(buf, rows, d):
    z = jnp.zeros((16,), jnp.float32)

    def row(r, _):
        for c in range(d // 16):
            buf[r, pl.ds(c * 16, 16)] = z
        return 0

    lax.fori_loop(0, rows, row, 0, unroll=False)


def _relu_add(dst, a, b, rows, d):
    def row(r, _):
        for c in range(d // 16):
            sl = pl.ds(c * 16, 16)
            dst[r, sl] = jnp.maximum(a[r, sl] + b[r, sl], 0.0)
        return 0

    lax.fori_loop(0, rows, row, 0, unroll=False)


def _sc_aggregate(table, src, dst, e):
    mesh = plsc.VectorSubcoreMesh(core_axis_name="c", subcore_axis_name="s",
                                  num_cores=NC, num_subcores=NS)

    @functools.partial(
        pl.kernel,
        out_type=jax.ShapeDtypeStruct((NC, N_PAD, D_IN), jnp.float32),
        mesh=mesh,
        scratch_types=[
            pltpu.VMEM((CB,), jnp.int32), pltpu.VMEM((CB,), jnp.int32),
            pltpu.VMEM((CB,), jnp.int32), pltpu.VMEM((CB,), jnp.int32),
            pltpu.VMEM((CB, D_IN), jnp.float32), pltpu.VMEM((CB, D_IN), jnp.float32),
            pltpu.VMEM((CB, D_IN), jnp.float32), pltpu.VMEM((CB, D_IN), jnp.float32),
            pltpu.VMEM_SHARED((N_PAD, D_IN), jnp.float32),
            pltpu.SemaphoreType.DMA, pltpu.SemaphoreType.DMA,
            pltpu.SemaphoreType.DMA, pltpu.SemaphoreType.DMA,
            pltpu.SemaphoreType.DMA, pltpu.SemaphoreType.DMA,
        ],
    )
    def k(tab_hbm, src_hbm, dst_hbm, e_hbm, out_hbm,
          sidx0, sidx1, didx0, didx1, rows0, rows1, e0, e1, acc_sh,
          sem_s0, sem_s1, sem_in0, sem_in1, sem_g0, sem_g1):
        cid = lax.axis_index("c")
        sid = lax.axis_index("s")
        gc = jnp.where(cid == 0, G0, G1)
        ebase = jnp.where(cid == 0, sid * (G0 * CB),
                          NS * (G0 * CB) + sid * (G1 * CB))

        sidx = (sidx0, sidx1)
        didx = (didx0, didx1)
        rows = (rows0, rows1)
        ebuf = (e0, e1)
        sem_s = (sem_s0, sem_s1)
        sem_in = (sem_in0, sem_in1)
        sem_g = (sem_g0, sem_g1)

        def issue_sidx(g, b):
            base = ebase + g * CB
            pltpu.async_copy(src_hbm.at[pl.ds(base, CB)], sidx[b], sem_s[b])

        def issue_de(g, b):
            base = ebase + g * CB
            pltpu.async_copy(dst_hbm.at[pl.ds(base, CB)], didx[b], sem_in[b])
            pltpu.async_copy(e_hbm.at[pl.ds(base, CB), :], ebuf[b], sem_in[b])

        def wait_sidx(b):
            pltpu.make_async_copy(src_hbm.at[pl.ds(0, CB)], sidx[b], sem_s[b]).wait()

        def issue_gather(b):
            pltpu.async_copy(tab_hbm.at[sidx[b]], rows[b], sem_g[b])

        def wait_gather(b):
            pltpu.make_async_copy(tab_hbm.at[sidx[b]], rows[b], sem_g[b]).wait()

        def wait_in(b):
            pltpu.make_async_copy(dst_hbm.at[pl.ds(0, CB)], didx[b], sem_in[b]).wait()
            pltpu.make_async_copy(e_hbm.at[pl.ds(0, CB), :], ebuf[b], sem_in[b]).wait()

        _zero_buf(rows0, CB, D_IN)
        row0 = sid * ROWS_PER_TILE
        for off, nr in WB_CHUNKS:
            pltpu.sync_copy(rows0.at[pl.ds(0, nr), :],
                            acc_sh.at[pl.ds(row0 + off, nr), :])
        plsc.subcore_barrier()

        issue_sidx(0, 0)
        issue_de(0, 0)
        issue_sidx(1, 1)
        issue_de(1, 1)
        wait_sidx(0)
        issue_gather(0)

        def step(t, _):
            for b in range(2):
                g = 2 * t + b
                nb = 1 - b

                @pl.when(g + 1 < gc)
                def _():
                    wait_sidx(nb)
                    issue_gather(nb)

                wait_gather(b)

                @pl.when(g + 2 < gc)
                def _():
                    issue_sidx(g + 2, b)

                wait_in(b)
                _relu_add(rows[b], rows[b], ebuf[b], CB, D_IN)
                pltpu.sync_copy(rows[b], acc_sh.at[didx[b]], add=True)

                @pl.when(g + 2 < gc)
                def _():
                    issue_de(g + 2, b)
            return 0

        lax.fori_loop(0, gc // 2, step, 0, unroll=False)
        plsc.subcore_barrier()

        for off, nr in WB_CHUNKS:
            sl = pl.ds(row0 + off, nr)
            st = pl.ds(0, nr)
            pltpu.sync_copy(acc_sh.at[sl, :], rows0.at[st, :])
            pltpu.sync_copy(rows0.at[st, :], out_hbm.at[cid, sl, :])

    return k(table, src, dst, e)


def _mlp1_body(x_ref, a0_ref, a1_ref, w1_ref, b1_ref, w2_ref, b2_ref, hh_ref):
    s = x_ref[...] + a0_ref[0] + a1_ref[0]
    t = jnp.maximum(jnp.dot(s, w1_ref[...], preferred_element_type=jnp.float32)
                    + b1_ref[...], 0.0)
    u = jnp.dot(t, w2_ref[...], preferred_element_type=jnp.float32) + b2_ref[...]
    h = jnp.maximum(u, 0.0)
    hh_ref[...] = jnp.concatenate([h, h], axis=1)


def _mlp1(x, acc, w1t, b1, w2t, b2):
    BN = 1000
    grid = (N // BN,)
    full = lambda shape: pl.BlockSpec(shape, lambda i: (0, 0))
    return pl.pallas_call(
        _mlp1_body,
        grid=grid,
        in_specs=[
            pl.BlockSpec((BN, D_IN), lambda i: (i, 0)),
            pl.BlockSpec((1, BN, D_IN), lambda i: (0, i, 0)),
            pl.BlockSpec((1, BN, D_IN), lambda i: (1, i, 0)),
            full((D_IN, LATENT)), full((1, LATENT)),
            full((LATENT, LATENT)), full((1, LATENT)),
        ],
        out_specs=pl.BlockSpec((BN, D_IN), lambda i: (i, 0)),
        out_shape=jax.ShapeDtypeStruct((N, D_IN), jnp.float32),
    )(x, acc, acc, w1t, b1, w2t, b2)


def _mlp23_body(hh_ref, a0_ref, a1_ref,
                mw1_ref, mb1_ref, mw2_ref, mb2_ref,
                lw1_ref, lb1_ref, lw2_ref, lb2_ref, mu_ref, ls_ref):
    h = hh_ref[:, :LATENT]
    a = a0_ref[0] + a1_ref[0]
    s2 = h + a[:, :LATENT]
    t2 = jnp.maximum(jnp.dot(s2, mw1_ref[...], preferred_element_type=jnp.float32)
                     + mb1_ref[...], 0.0)
    mu_ref[...] = jnp.dot(t2, mw2_ref[...], preferred_element_type=jnp.float32) + mb2_ref[...]
    s3 = h + a[:, LATENT:]
    t3 = jnp.maximum(jnp.dot(s3, lw1_ref[...], preferred_element_type=jnp.float32)
                     + lb1_ref[...], 0.0)
    u3 = jnp.dot(t3, lw2_ref[...], preferred_element_type=jnp.float32) + lb2_ref[...]
    ls_ref[...] = jnp.clip(u3, -10.0, 10.0)


def _mlp23(hh, acc, mw1t, mb1, mw2t, mb2, lw1t, lb1, lw2t, lb2):
    BN = 1000
    grid = (N // BN,)
    full = lambda shape: pl.BlockSpec(shape, lambda i: (0, 0))
    return pl.pallas_call(
        _mlp23_body,
        grid=grid,
        in_specs=[
            pl.BlockSpec((BN, D_IN), lambda i: (i, 0)),
            pl.BlockSpec((1, BN, D_IN), lambda i: (0, i, 0)),
            pl.BlockSpec((1, BN, D_IN), lambda i: (1, i, 0)),
            full((LATENT, LATENT)), full((1, LATENT)),
            full((LATENT, LATENT)), full((1, LATENT)),
            full((LATENT, LATENT)), full((1, LATENT)),
            full((LATENT, LATENT)), full((1, LATENT)),
        ],
        out_specs=[
            pl.BlockSpec((BN, LATENT), lambda i: (i, 0)),
            pl.BlockSpec((BN, LATENT), lambda i: (i, 0)),
        ],
        out_shape=[
            jax.ShapeDtypeStruct((N, LATENT), jnp.float32),
            jax.ShapeDtypeStruct((N, LATENT), jnp.float32),
        ],
    )(hh, acc, acc, mw1t, mb1, mw2t, mb2, lw1t, lb1, lw2t, lb2)


def kernel(x, edge_index, edge_attr,
           le1_W, le1_b, c1_W1, c1_b1, c1_W2, c1_b2,
           le2_W, le2_b, mu_W1, mu_b1, mu_W2, mu_b2,
           le3_W, le3_b, ls_W1, ls_b1, ls_W2, ls_b2):
    src = edge_index[0].astype(jnp.int32)
    dst = edge_index[1].astype(jnp.int32)
    pad = E_PAD - E
    src = jnp.pad(src, (0, pad))
    dst = jnp.pad(dst, (0, pad), constant_values=N)
    ea = jnp.pad(edge_attr, ((0, pad), (0, 0)))

    w23t = jnp.concatenate([le2_W.T, le3_W.T], axis=1)
    b23 = jnp.concatenate([le2_b, le3_b]).reshape(1, -1)
    e1 = _edge_embed(ea, le1_W.T, le1_b.reshape(1, -1))
    e23 = _edge_embed(ea, w23t, b23)

    acc1 = _sc_aggregate(x, src, dst, e1)
    hh = _mlp1(x, acc1, c1_W1.T, c1_b1.reshape(1, -1), c1_W2.T, c1_b2.reshape(1, -1))
    acc23 = _sc_aggregate(hh, src, dst, e23)
    mu, logstd = _mlp23(
        hh, acc23,
        mu_W1.T, mu_b1.reshape(1, -1), mu_W2.T, mu_b2.reshape(1, -1),
        ls_W1.T, ls_b1.reshape(1, -1), ls_W2.T, ls_b2.reshape(1, -1),
    )
    return (mu, logstd)


# speedup vs baseline: 1.2234x; 1.1274x over previous
import functools

import jax
import jax.numpy as jnp
from jax import lax
from jax.experimental import pallas as pl
from jax.experimental.pallas import tpu as pltpu
from jax.experimental.pallas import tpu_sc as plsc

N = 10000
E = 320000
D_IN = 128
D_EDGE = 16
LATENT = 64

NC = 2
NS = 16
NW = NC * NS
CB = 96
G0 = 148
G1 = 64
E_PAD = NS * (G0 + G1) * CB
N_PAD = 10112
ROWS_PER_TILE = N_PAD // NS
WB_CHUNKS = [(j * CB, CB) for j in range(ROWS_PER_TILE // CB)]
if ROWS_PER_TILE % CB:
    WB_CHUNKS.append((ROWS_PER_TILE - ROWS_PER_TILE % CB, ROWS_PER_TILE % CB))


def _edge_embed_body(ea_ref, w1_ref, b1_ref, w23_ref, b23_ref, e1_ref, e23_ref):
    ea = ea_ref[...]
    e1_ref[...] = jnp.dot(ea, w1_ref[...], preferred_element_type=jnp.float32) + b1_ref[...]
    e23_ref[...] = jnp.dot(ea, w23_ref[...], preferred_element_type=jnp.float32) + b23_ref[...]


def _edge_embed(ea_pad, w1t, b1, w23t, b23):
    BE = 2048
    grid = (E_PAD // BE,)
    full = lambda shape: pl.BlockSpec(shape, lambda i: (0, 0))
    return pl.pallas_call(
        _edge_embed_body,
        grid=grid,
        in_specs=[
            pl.BlockSpec((BE, D_EDGE), lambda i: (i, 0)),
            full((D_EDGE, D_IN)), full((1, D_IN)),
            full((D_EDGE, D_IN)), full((1, D_IN)),
        ],
        out_specs=[
            pl.BlockSpec((BE, D_IN), lambda i: (i, 0)),
            pl.BlockSpec((BE, D_IN), lambda i: (i, 0)),
        ],
        out_shape=[
            jax.ShapeDtypeStruct((E_PAD, D_IN), jnp.float32),
            jax.ShapeDtypeStruct((E_PAD, D_IN), jnp.float32),
        ],
    )(ea_pad, w1t, b1, w23t, b23)


def _zero_buf(buf, rows, d):
    z = jnp.zeros((16,), jnp.float32)

    def row(r, _):
        for c in range(d // 16):
            buf[r, pl.ds(c * 16, 16)] = z
        return 0

    lax.fori_loop(0, rows, row, 0, unroll=False)


def _relu_add(dst, a, b, rows, d):
    def row(r, _):
        for c in range(d // 16):
            sl = pl.ds(c * 16, 16)
            dst[r, sl] = jnp.maximum(a[r, sl] + b[r, sl], 0.0)
        return 0

    lax.fori_loop(0, rows, row, 0, unroll=False)


def _sc_aggregate(table, src, dst, e):
    mesh = plsc.VectorSubcoreMesh(core_axis_name="c", subcore_axis_name="s",
                                  num_cores=NC, num_subcores=NS)

    @functools.partial(
        pl.kernel,
        out_type=jax.ShapeDtypeStruct((NC, N_PAD, D_IN), jnp.float32),
        mesh=mesh,
        scratch_types=[
            pltpu.VMEM((CB,), jnp.int32), pltpu.VMEM((CB,), jnp.int32),
            pltpu.VMEM((CB,), jnp.int32), pltpu.VMEM((CB,), jnp.int32),
            pltpu.VMEM((CB, D_IN), jnp.float32), pltpu.VMEM((CB, D_IN), jnp.float32),
            pltpu.VMEM((CB, D_IN), jnp.float32), pltpu.VMEM((CB, D_IN), jnp.float32),
            pltpu.VMEM_SHARED((N_PAD, D_IN), jnp.float32),
            pltpu.SemaphoreType.DMA, pltpu.SemaphoreType.DMA,
            pltpu.SemaphoreType.DMA, pltpu.SemaphoreType.DMA,
            pltpu.SemaphoreType.DMA, pltpu.SemaphoreType.DMA,
        ],
    )
    def k(tab_hbm, src_hbm, dst_hbm, e_hbm, out_hbm,
          sidx0, sidx1, didx0, didx1, rows0, rows1, e0, e1, acc_sh,
          sem_s0, sem_s1, sem_in0, sem_in1, sem_g0, sem_g1):
        cid = lax.axis_index("c")
        sid = lax.axis_index("s")
        gc = jnp.where(cid == 0, G0, G1)
        ebase = jnp.where(cid == 0, sid * (G0 * CB),
                          NS * (G0 * CB) + sid * (G1 * CB))

        sidx = (sidx0, sidx1)
        didx = (didx0, didx1)
        rows = (rows0, rows1)
        ebuf = (e0, e1)
        sem_s = (sem_s0, sem_s1)
        sem_in = (sem_in0, sem_in1)
        sem_g = (sem_g0, sem_g1)

        def issue_sidx(g, b):
            base = ebase + g * CB
            pltpu.async_copy(src_hbm.at[pl.ds(base, CB)], sidx[b], sem_s[b])

        def issue_de(g, b):
            base = ebase + g * CB
            pltpu.async_copy(dst_hbm.at[pl.ds(base, CB)], didx[b], sem_in[b])
            pltpu.async_copy(e_hbm.at[pl.ds(base, CB), :], ebuf[b], sem_in[b])

        def wait_sidx(b):
            pltpu.make_async_copy(src_hbm.at[pl.ds(0, CB)], sidx[b], sem_s[b]).wait()

        def issue_gather(b):
            pltpu.async_copy(tab_hbm.at[sidx[b]], rows[b], sem_g[b])

        def wait_gather(b):
            pltpu.make_async_copy(tab_hbm.at[sidx[b]], rows[b], sem_g[b]).wait()

        def wait_in(b):
            pltpu.make_async_copy(dst_hbm.at[pl.ds(0, CB)], didx[b], sem_in[b]).wait()
            pltpu.make_async_copy(e_hbm.at[pl.ds(0, CB), :], ebuf[b], sem_in[b]).wait()

        _zero_buf(rows0, CB, D_IN)
        row0 = sid * ROWS_PER_TILE
        for off, nr in WB_CHUNKS:
            pltpu.sync_copy(rows0.at[pl.ds(0, nr), :],
                            acc_sh.at[pl.ds(row0 + off, nr), :])
        plsc.subcore_barrier()

        issue_sidx(0, 0)
        issue_de(0, 0)
        issue_sidx(1, 1)
        issue_de(1, 1)
        wait_sidx(0)
        issue_gather(0)

        def step(t, _):
            for b in range(2):
                g = 2 * t + b
                nb = 1 - b

                @pl.when(g + 1 < gc)
                def _():
                    wait_sidx(nb)
                    issue_gather(nb)

                wait_gather(b)

                @pl.when(g + 2 < gc)
                def _():
                    issue_sidx(g + 2, b)

                wait_in(b)
                _relu_add(rows[b], rows[b], ebuf[b], CB, D_IN)
                pltpu.sync_copy(rows[b], acc_sh.at[didx[b]], add=True)

                @pl.when(g + 2 < gc)
                def _():
                    issue_de(g + 2, b)
            return 0

        lax.fori_loop(0, gc // 2, step, 0, unroll=False)
        plsc.subcore_barrier()

        for off, nr in WB_CHUNKS:
            sl = pl.ds(row0 + off, nr)
            st = pl.ds(0, nr)
            pltpu.sync_copy(acc_sh.at[sl, :], rows0.at[st, :])
            pltpu.sync_copy(rows0.at[st, :], out_hbm.at[cid, sl, :])

    return k(table, src, dst, e)


def _mlp1_body(x_ref, a0_ref, a1_ref, w1_ref, b1_ref, w2_ref, b2_ref, hh_ref):
    s = x_ref[...] + a0_ref[0] + a1_ref[0]
    t = jnp.maximum(jnp.dot(s, w1_ref[...], preferred_element_type=jnp.float32)
                    + b1_ref[...], 0.0)
    u = jnp.dot(t, w2_ref[...], preferred_element_type=jnp.float32) + b2_ref[...]
    h = jnp.maximum(u, 0.0)
    hh_ref[...] = jnp.concatenate([h, h], axis=1)


def _mlp1(x, acc, w1t, b1, w2t, b2):
    BN = 1000
    grid = (N // BN,)
    full = lambda shape: pl.BlockSpec(shape, lambda i: (0, 0))
    return pl.pallas_call(
        _mlp1_body,
        grid=grid,
        in_specs=[
            pl.BlockSpec((BN, D_IN), lambda i: (i, 0)),
            pl.BlockSpec((1, BN, D_IN), lambda i: (0, i, 0)),
            pl.BlockSpec((1, BN, D_IN), lambda i: (1, i, 0)),
            full((D_IN, LATENT)), full((1, LATENT)),
            full((LATENT, LATENT)), full((1, LATENT)),
        ],
        out_specs=pl.BlockSpec((BN, D_IN), lambda i: (i, 0)),
        out_shape=jax.ShapeDtypeStruct((N, D_IN), jnp.float32),
    )(x, acc, acc, w1t, b1, w2t, b2)


def _mlp23_body(hh_ref, a0_ref, a1_ref,
                mw1_ref, mb1_ref, mw2_ref, mb2_ref,
                lw1_ref, lb1_ref, lw2_ref, lb2_ref, mu_ref, ls_ref):
    h = hh_ref[:, :LATENT]
    a = a0_ref[0] + a1_ref[0]
    s2 = h + a[:, :LATENT]
    t2 = jnp.maximum(jnp.dot(s2, mw1_ref[...], preferred_element_type=jnp.float32)
                     + mb1_ref[...], 0.0)
    mu_ref[...] = jnp.dot(t2, mw2_ref[...], preferred_element_type=jnp.float32) + mb2_ref[...]
    s3 = h + a[:, LATENT:]
    t3 = jnp.maximum(jnp.dot(s3, lw1_ref[...], preferred_element_type=jnp.float32)
                     + lb1_ref[...], 0.0)
    u3 = jnp.dot(t3, lw2_ref[...], preferred_element_type=jnp.float32) + lb2_ref[...]
    ls_ref[...] = jnp.clip(u3, -10.0, 10.0)


def _mlp23(hh, acc, mw1t, mb1, mw2t, mb2, lw1t, lb1, lw2t, lb2):
    BN = 1000
    grid = (N // BN,)
    full = lambda shape: pl.BlockSpec(shape, lambda i: (0, 0))
    return pl.pallas_call(
        _mlp23_body,
        grid=grid,
        in_specs=[
            pl.BlockSpec((BN, D_IN), lambda i: (i, 0)),
            pl.BlockSpec((1, BN, D_IN), lambda i: (0, i, 0)),
            pl.BlockSpec((1, BN, D_IN), lambda i: (1, i, 0)),
            full((LATENT, LATENT)), full((1, LATENT)),
            full((LATENT, LATENT)), full((1, LATENT)),
            full((LATENT, LATENT)), full((1, LATENT)),
            full((LATENT, LATENT)), full((1, LATENT)),
        ],
        out_specs=[
            pl.BlockSpec((BN, LATENT), lambda i: (i, 0)),
            pl.BlockSpec((BN, LATENT), lambda i: (i, 0)),
        ],
        out_shape=[
            jax.ShapeDtypeStruct((N, LATENT), jnp.float32),
            jax.ShapeDtypeStruct((N, LATENT), jnp.float32),
        ],
    )(hh, acc, acc, mw1t, mb1, mw2t, mb2, lw1t, lb1, lw2t, lb2)


def kernel(x, edge_index, edge_attr,
           le1_W, le1_b, c1_W1, c1_b1, c1_W2, c1_b2,
           le2_W, le2_b, mu_W1, mu_b1, mu_W2, mu_b2,
           le3_W, le3_b, ls_W1, ls_b1, ls_W2, ls_b2):
    src = edge_index[0].astype(jnp.int32)
    dst = edge_index[1].astype(jnp.int32)
    pad = E_PAD - E
    src = jnp.pad(src, (0, pad))
    dst = jnp.pad(dst, (0, pad), constant_values=N)
    ea = jnp.pad(edge_attr, ((0, pad), (0, 0)))

    w23t = jnp.concatenate([le2_W.T, le3_W.T], axis=1)
    b23 = jnp.concatenate([le2_b, le3_b]).reshape(1, -1)
    e1, e23 = _edge_embed(ea, le1_W.T, le1_b.reshape(1, -1), w23t, b23)

    acc1 = _sc_aggregate(x, src, dst, e1)
    hh = _mlp1(x, acc1, c1_W1.T, c1_b1.reshape(1, -1), c1_W2.T, c1_b2.reshape(1, -1))
    acc23 = _sc_aggregate(hh, src, dst, e23)
    mu, logstd = _mlp23(
        hh, acc23,
        mu_W1.T, mu_b1.reshape(1, -1), mu_W2.T, mu_b2.reshape(1, -1),
        ls_W1.T, ls_b1.reshape(1, -1), ls_W2.T, ls_b2.reshape(1, -1),
    )
    return (mu, logstd)
